# Initial kernel scaffold; baseline (speedup 1.0000x reference)
import jax
import jax.numpy as jnp
from jax.experimental import pallas as pl


def kernel(obs, table, W, b):
    raise NotImplementedError("write your pallas kernel here")


# speedup vs baseline: 6.6605x; 6.6605x over previous
import functools

import jax
import jax.numpy as jnp
from jax import lax
from jax.experimental import pallas as pl
from jax.experimental.pallas import tpu as pltpu
from jax.experimental.pallas import tpu_sc as plsc

NC, NS = 2, 16
NW = NC * NS
GROUP = 128
K = 8


def _sc_gather(table, idx2d, n_rows, d):
    n_groups = n_rows // GROUP
    groups_per_w = n_groups // NW
    steps = groups_per_w // K
    mesh = plsc.VectorSubcoreMesh(core_axis_name="c", subcore_axis_name="s")

    @functools.partial(
        pl.kernel,
        out_type=jax.ShapeDtypeStruct((n_rows, d), jnp.float32),
        mesh=mesh,
        scratch_types=[
            pltpu.VMEM((K, GROUP), jnp.int32),
            pltpu.VMEM((K * GROUP, d), jnp.float32),
            pltpu.SemaphoreType.DMA,
        ],
        compiler_params=pltpu.CompilerParams(use_tc_tiling_on_sc=False),
    )
    def gather_kernel(table_hbm, idx_hbm, out_hbm, idx_v, rows_v, sem):
        wid = lax.axis_index("s") * NC + lax.axis_index("c")
        g0 = wid * groups_per_w

        def step(t, carry):
            g = g0 + t * K
            pltpu.sync_copy(idx_hbm.at[pl.ds(g, K)], idx_v)
            copies = [
                pltpu.async_copy(
                    table_hbm.at[idx_v.at[j]],
                    rows_v.at[pl.ds(j * GROUP, GROUP)],
                    sem,
                )
                for j in range(K)
            ]
            for c in copies:
                c.wait()
            pltpu.sync_copy(rows_v, out_hbm.at[pl.ds(g * GROUP, K * GROUP)])
            return carry

        lax.fori_loop(0, steps, step, 0)

    return gather_kernel(table, idx2d)


def _tc_matmul(x, w, b):
    n, f = x.shape
    out_dim = w.shape[0]
    bm = 1024

    def mm_kernel(x_ref, w_ref, b_ref, o_ref):
        acc = lax.dot_general(
            x_ref[...], w_ref[...], (((1,), (1,)), ((), ())),
            preferred_element_type=jnp.float32,
        )
        o_ref[...] = acc + b_ref[...]

    return pl.pallas_call(
        mm_kernel,
        grid=(n // bm,),
        in_specs=[
            pl.BlockSpec((bm, f), lambda i: (i, 0)),
            pl.BlockSpec((out_dim, f), lambda i: (0, 0)),
            pl.BlockSpec((1, out_dim), lambda i: (0, 0)),
        ],
        out_specs=pl.BlockSpec((bm, out_dim), lambda i: (i, 0)),
        out_shape=jax.ShapeDtypeStruct((n, out_dim), jnp.float32),
    )(x, w, b.reshape(1, out_dim))


def kernel(obs, table, W, b):
    batch, context_len, n_agents, features = obs.shape
    n = batch * context_len * n_agents
    d = table.shape[1]
    n_idx = n * features
    idx2d = obs.reshape(n_idx // GROUP, GROUP)
    gathered = _sc_gather(table, idx2d, n_idx, d)
    x = gathered.reshape(n, features * d)
    out = _tc_matmul(x, W, b)
    return out.reshape(batch, context_len, n_agents, -1)


# speedup vs baseline: 6.6610x; 1.0001x over previous
import functools

import jax
import jax.numpy as jnp
from jax import lax
from jax.experimental import pallas as pl
from jax.experimental.pallas import tpu as pltpu
from jax.experimental.pallas import tpu_sc as plsc

NC, NS = 2, 16
NW = NC * NS
GROUP = 128
K = 8


def _sc_gather(table, idx_flat, n_idx, d):
    idx_per_w = n_idx // NW
    chunk = GROUP * K
    steps = idx_per_w // chunk
    mesh = plsc.VectorSubcoreMesh(core_axis_name="c", subcore_axis_name="s")

    @functools.partial(
        pl.kernel,
        out_type=jax.ShapeDtypeStruct((n_idx, d), jnp.float32),
        mesh=mesh,
        scratch_types=[
            pltpu.VMEM((chunk,), jnp.int32),
            pltpu.VMEM((chunk, d), jnp.float32),
            pltpu.SemaphoreType.DMA,
        ],
        compiler_params=pltpu.CompilerParams(use_tc_tiling_on_sc=False),
    )
    def gather_kernel(table_hbm, idx_hbm, out_hbm, idx_v, buf_v, sem):
        wid = lax.axis_index("s") * NC + lax.axis_index("c")
        i0 = wid * idx_per_w

        def step(t, carry):
            base = i0 + t * chunk
            pltpu.sync_copy(idx_hbm.at[pl.ds(base, chunk)], idx_v)
            copies = [
                pltpu.async_copy(
                    table_hbm.at[idx_v.at[pl.ds(j * GROUP, GROUP)]],
                    buf_v.at[pl.ds(j * GROUP, GROUP)],
                    sem,
                )
                for j in range(K)
            ]
            for c in copies:
                c.wait()
            pltpu.sync_copy(buf_v, out_hbm.at[pl.ds(base, chunk)])
            return carry

        lax.fori_loop(0, steps, step, 0)

    return gather_kernel(table, idx_flat)


def _tc_matmul(x, w, b):
    n, f = x.shape
    out_dim = w.shape[0]
    bm = 1024

    def mm_kernel(x_ref, w_ref, b_ref, o_ref):
        acc = lax.dot_general(
            x_ref[...], w_ref[...], (((1,), (1,)), ((), ())),
            preferred_element_type=jnp.float32,
        )
        o_ref[...] = acc + b_ref[...]

    return pl.pallas_call(
        mm_kernel,
        grid=(n // bm,),
        in_specs=[
            pl.BlockSpec((bm, f), lambda i: (i, 0)),
            pl.BlockSpec((out_dim, f), lambda i: (0, 0)),
            pl.BlockSpec((1, out_dim), lambda i: (0, 0)),
        ],
        out_specs=pl.BlockSpec((bm, out_dim), lambda i: (i, 0)),
        out_shape=jax.ShapeDtypeStruct((n, out_dim), jnp.float32),
    )(x, w, b.reshape(1, out_dim))


def kernel(obs, table, W, b):
    batch, context_len, n_agents, features = obs.shape
    n = batch * context_len * n_agents
    d = table.shape[1]
    idx_flat = obs.reshape(-1)
    gathered = _sc_gather(table, idx_flat, n * features, d)
    x = gathered.reshape(n, features * d)
    out = _tc_matmul(x, W, b)
    return out.reshape(batch, context_len, n_agents, -1)


# speedup vs baseline: 6.9559x; 1.0443x over previous
import functools

import jax
import jax.numpy as jnp
from jax import lax
from jax.experimental import pallas as pl
from jax.experimental.pallas import tpu as pltpu
from jax.experimental.pallas import tpu_sc as plsc

NC, NS = 2, 16
NW = NC * NS
CR = 64
FEATS = 26
CIDX = CR * FEATS
NSTREAM = CIDX // 128


def _compact(idx_v, ic_v):
    lane = lax.iota(jnp.int32, 16)
    for j in range(CIDX // 16):
        flat = lane + (16 * j)
        ri = flat // FEATS
        ci = flat - ri * FEATS
        ic_v[pl.ds(16 * j, 16)] = plsc.load_gather(idx_v, [ri, ci])


def _sc_gather(table, idx2d, n_rows, d):
    rows_per_w = n_rows // NW
    steps = rows_per_w // CR
    mesh = plsc.VectorSubcoreMesh(core_axis_name="c", subcore_axis_name="s")

    @functools.partial(
        pl.kernel,
        out_type=jax.ShapeDtypeStruct((n_rows * FEATS, d), jnp.float32),
        mesh=mesh,
        scratch_types=[
            pltpu.VMEM((CR, FEATS), jnp.int32),
            pltpu.VMEM((CR, FEATS), jnp.int32),
            pltpu.VMEM((CIDX,), jnp.int32),
            pltpu.VMEM((CIDX,), jnp.int32),
            pltpu.VMEM((CIDX, d), jnp.float32),
            pltpu.VMEM((CIDX, d), jnp.float32),
            pltpu.SemaphoreType.DMA,
            pltpu.SemaphoreType.DMA,
            pltpu.SemaphoreType.DMA,
        ],
        compiler_params=pltpu.CompilerParams(
            use_tc_tiling_on_sc=False, needs_layout_passes=False
        ),
    )
    def gather_kernel(table_hbm, idx_hbm, out_hbm,
                      idx_a, idx_b, ic_a, ic_b, buf_a, buf_b,
                      sem_g, sem_wa, sem_wb):
        wid = lax.axis_index("s") * NC + lax.axis_index("c")
        w0 = wid * rows_per_w

        bufs = ((idx_a, ic_a, buf_a, sem_wa), (idx_b, ic_b, buf_b, sem_wb))

        def body(t2, carry):
            for p, (idx_v, ic_v, buf_v, sem_w) in enumerate(bufs):
                r0 = w0 + (2 * t2 + p) * CR
                o0 = r0 * FEATS
                pltpu.sync_copy(idx_hbm.at[pl.ds(r0, CR)], idx_v)
                _compact(idx_v, ic_v)

                @pl.when(t2 > 0)
                def _():
                    pltpu.make_async_copy(
                        buf_v, out_hbm.at[pl.ds(o0, CIDX)], sem_w
                    ).wait()

                copies = [
                    pltpu.async_copy(
                        table_hbm.at[ic_v.at[pl.ds(128 * s, 128)]],
                        buf_v.at[pl.ds(128 * s, 128)],
                        sem_g,
                    )
                    for s in range(NSTREAM)
                ]
                for c in copies:
                    c.wait()
                pltpu.async_copy(buf_v, out_hbm.at[pl.ds(o0, CIDX)], sem_w)
            return carry

        lax.fori_loop(0, steps // 2, body, 0)
        for _, _, buf_v, sem_w in bufs:
            pltpu.make_async_copy(
                buf_v, out_hbm.at[pl.ds(w0 * FEATS, CIDX)], sem_w
            ).wait()

    return gather_kernel(table, idx2d)


def _tc_matmul(x, w, b):
    n, f = x.shape
    out_dim = w.shape[0]
    bm = 1024

    def mm_kernel(x_ref, w_ref, b_ref, o_ref):
        acc = lax.dot_general(
            x_ref[...], w_ref[...], (((1,), (1,)), ((), ())),
            preferred_element_type=jnp.float32,
        )
        o_ref[...] = acc + b_ref[...]

    return pl.pallas_call(
        mm_kernel,
        grid=(n // bm,),
        in_specs=[
            pl.BlockSpec((bm, f), lambda i: (i, 0)),
            pl.BlockSpec((out_dim, f), lambda i: (0, 0)),
            pl.BlockSpec((1, out_dim), lambda i: (0, 0)),
        ],
        out_specs=pl.BlockSpec((bm, out_dim), lambda i: (i, 0)),
        out_shape=jax.ShapeDtypeStruct((n, out_dim), jnp.float32),
    )(x, w, b.reshape(1, out_dim))


def kernel(obs, table, W, b):
    batch, context_len, n_agents, features = obs.shape
    n = batch * context_len * n_agents
    d = table.shape[1]
    idx2d = obs.reshape(n, features)
    gathered = _sc_gather(table, idx2d, n, d)
    x = gathered.reshape(n, features * d)
    out = _tc_matmul(x, W, b)
    return out.reshape(batch, context_len, n_agents, -1)
